# baseline (speedup 1.0000x reference)
import functools

import jax
import jax.numpy as jnp
from jax import lax
from jax.experimental import pallas as pl
from jax.experimental.pallas import tpu as pltpu
from jax.experimental.pallas import tpu_sc as plsc

_N_GRID = 100000
_EMB = 128
_RDIM = 32
_E = 320000

_NW = 32
_EPW = _E // _NW
_IW = 80
_KSUB = 5
_CH = _IW * _KSUB
_NCHUNK = _EPW // _CH

_BM = 1000
_NSEC = 4
_QROWS = _N_GRID // _NSEC


def _project(table, w_r):
    def body(x0, x1, x2, x3, w_ref, p_ref):
        for j, x in enumerate((x0, x1, x2, x3)):
            p_ref[:, j * _RDIM:(j + 1) * _RDIM] = jnp.dot(
                x[...], w_ref[...], preferred_element_type=jnp.float32)

    nblk = _QROWS // _BM
    return pl.pallas_call(
        body,
        grid=(nblk,),
        in_specs=[
            pl.BlockSpec((_BM, _EMB), lambda i, j=j: (j * nblk + i, 0))
            for j in range(_NSEC)
        ] + [pl.BlockSpec((_EMB, _RDIM), lambda i: (0, 0))],
        out_specs=pl.BlockSpec((_BM, _EMB), lambda i: (i, 0)),
        out_shape=jax.ShapeDtypeStruct((_QROWS, _EMB), jnp.float32),
    )(table, table, table, table, w_r)


def _remap_idx(h2, p2, n2):
    def body(h_ref, p_ref, n_ref, ho_ref, po_ref, no_ref):
        for src, dst in ((h_ref, ho_ref), (p_ref, po_ref), (n_ref, no_ref)):
            r = src[...]
            j = ((r >= _QROWS).astype(jnp.int32)
                 + (r >= 2 * _QROWS).astype(jnp.int32)
                 + (r >= 3 * _QROWS).astype(jnp.int32))
            dst[...] = 4 * r - (_N_GRID - 1) * j

    spec = pl.BlockSpec(h2.shape, lambda: (0, 0))
    out = jax.ShapeDtypeStruct(h2.shape, jnp.int32)
    return pl.pallas_call(
        body,
        in_specs=[spec] * 3,
        out_specs=[spec] * 3,
        out_shape=[out] * 3,
    )(h2, p2, n2)


def _edge_z(p_tab, r_vec, h1, tp1, tn1):
    mesh = plsc.VectorSubcoreMesh(core_axis_name="c", subcore_axis_name="s")

    row_t = pltpu.VMEM((_CH, _RDIM), jnp.float32)
    idx_t = pltpu.VMEM((_CH,), jnp.int32)

    @functools.partial(
        pl.kernel,
        mesh=mesh,
        compiler_params=pltpu.CompilerParams(
            needs_layout_passes=False, use_tc_tiling_on_sc=False),
        out_type=jax.ShapeDtypeStruct((_E,), jnp.float32),
        scratch_types=(
            [idx_t] * 9
            + [row_t] * 9
            + [
                pltpu.VMEM((_CH,), jnp.float32),
                pltpu.VMEM((_RDIM,), jnp.float32),
            ]
            + [pltpu.SemaphoreType.DMA] * 6
        ),
    )
    def kern(p_hbm, r_hbm, h_hbm, tp_hbm, tn_hbm, z_hbm,
             hi0, pi0, ni0, hi1, pi1, ni1, hi2, pi2, ni2,
             ha, pa, na, hb, pb, nb, hc, pc, nc, zv, rbuf,
             sga, sgb, sgc, sia, sib, sic):
        idx_sets = [(hi0, pi0, ni0, sia), (hi1, pi1, ni1, sib),
                    (hi2, pi2, ni2, sic)]
        row_sets = [(ha, pa, na, sga), (hb, pb, nb, sgb), (hc, pc, nc, sgc)]

        wid = lax.axis_index("s") * 2 + lax.axis_index("c")
        ebase = wid * _EPW
        pltpu.sync_copy(r_hbm, rbuf)

        def idx_copies(c, s):
            hi, pi, ni, sem = idx_sets[s]
            src = pl.ds(ebase + c * _CH, _CH)
            return [(h_hbm.at[src], hi, sem), (tp_hbm.at[src], pi, sem),
                    (tn_hbm.at[src], ni, sem)]

        def gather_copies(s):
            hi, pi, ni, _ = idx_sets[s]
            hr, pr, nr, sem = row_sets[s]
            out = []
            for j in range(_KSUB):
                sl = pl.ds(j * _IW, _IW)
                out.append((p_hbm.at[hi.at[sl]], hr.at[sl], sem))
                out.append((p_hbm.at[pi.at[sl]], pr.at[sl], sem))
                out.append((p_hbm.at[ni.at[sl]], nr.at[sl], sem))
            return out

        def issue(copies):
            for s, d, sm in copies:
                pltpu.async_copy(s, d, sm)

        def drain(copies):
            for s, d, sm in copies:
                pltpu.make_async_copy(s, d, sm).wait()

        def compute(c, s):
            hr, pr, nr, _ = row_sets[s]

            def group(g, carry2):
                lane = lax.iota(jnp.int32, 16)
                ridx = lane + g * 16
                g1 = jnp.zeros((16,), jnp.float32)
                g2 = jnp.zeros((16,), jnp.float32)
                cr = jnp.zeros((16,), jnp.float32)
                for d in range(_RDIM):
                    cidx = (lane + d) & (_RDIM - 1)
                    hd = plsc.load_gather(hr, [ridx, cidx])
                    pd = plsc.load_gather(pr, [ridx, cidx])
                    nd = plsc.load_gather(nr, [ridx, cidx])
                    rv = plsc.load_gather(rbuf, [cidx])
                    u = hd - pd
                    v = hd - nd
                    g1 = g1 + u * u
                    g2 = g2 + v * v
                    cr = cr + rv * (u - v)
                zv[pl.ds(g * 16, 16)] = g1 - g2 + cr + cr
                return carry2

            lax.fori_loop(0, _CH // 16, group, 0)
            pltpu.sync_copy(zv, z_hbm.at[pl.ds(ebase + c * _CH, _CH)])

        def body(c, s, with_idx, with_gather):
            drain(gather_copies(s))
            if with_idx:
                issue(idx_copies(c + 3, s))
            if with_gather:
                drain(idx_copies(c + 2, (s + 2) % 3))
                issue(gather_copies((s + 2) % 3))
            compute(c, s)

        issue(idx_copies(0, 0))
        issue(idx_copies(1, 1))
        issue(idx_copies(2, 2))
        drain(idx_copies(0, 0))
        issue(gather_copies(0))
        drain(idx_copies(1, 1))
        issue(gather_copies(1))

        def triple(k, carry):
            c0 = 3 * k
            for j in range(3):
                body(c0 + j, j, True, True)
            return carry

        lax.fori_loop(0, (_NCHUNK - 4) // 3, triple, 0)
        body(_NCHUNK - 4, 0, True, True)
        body(_NCHUNK - 3, 1, False, True)
        body(_NCHUNK - 2, 2, False, False)
        body(_NCHUNK - 1, 0, False, False)

    return kern(p_tab, r_vec, h1, tp1, tn1)


def _softplus_sum(z2d):
    def body(z_ref, o_ref):
        x = z_ref[...]
        sp = jnp.maximum(x, 0.0) + jnp.log1p(jnp.exp(-jnp.abs(x)))
        o_ref[...] = jnp.sum(sp)[None, None]

    return pl.pallas_call(
        body,
        in_specs=[pl.BlockSpec(z2d.shape, lambda: (0, 0))],
        out_specs=pl.BlockSpec((1, 1), lambda: (0, 0)),
        out_shape=jax.ShapeDtypeStruct((1, 1), jnp.float32),
    )(z2d)


def kernel(city_grid_embedding, graph_relation_embed, graph_W_R,
           h, t_pos, t_neg, city_id, relation):
    w_r = graph_W_R[relation]
    r_embed = graph_relation_embed[relation]

    p_tab = _project(city_grid_embedding, w_r).reshape(_N_GRID, _RDIM)

    h2, p2, n2 = (x.astype(jnp.int32).reshape(_E // 128, 128)
                  for x in (h, t_pos, t_neg))
    hm, pm, nm = _remap_idx(h2, p2, n2)

    z = _edge_z(p_tab, r_embed,
                hm.reshape(_E), pm.reshape(_E), nm.reshape(_E))

    loss = _softplus_sum(z.reshape(_E // 128, 128))
    return loss[0, 0]


# speedup vs baseline: 1.0194x; 1.0194x over previous
import functools

import jax
import jax.numpy as jnp
from jax import lax
from jax.experimental import pallas as pl
from jax.experimental.pallas import tpu as pltpu
from jax.experimental.pallas import tpu_sc as plsc

_N_GRID = 100000
_EMB = 128
_RDIM = 32
_E = 320000

_NW = 32
_EPW = _E // _NW
_IW = 80
_KSUB = 5
_CH = _IW * _KSUB
_NCHUNK = _EPW // _CH

_BM = 1000
_NSEC = 4
_QROWS = _N_GRID // _NSEC


def _project(table, w_r):
    def body(x0, x1, x2, x3, w_ref, p_ref):
        for j, x in enumerate((x0, x1, x2, x3)):
            p_ref[:, j * _RDIM:(j + 1) * _RDIM] = jnp.dot(
                x[...], w_ref[...], preferred_element_type=jnp.float32)

    nblk = _QROWS // _BM
    return pl.pallas_call(
        body,
        grid=(nblk,),
        in_specs=[
            pl.BlockSpec((_BM, _EMB), lambda i, j=j: (j * nblk + i, 0))
            for j in range(_NSEC)
        ] + [pl.BlockSpec((_EMB, _RDIM), lambda i: (0, 0))],
        out_specs=pl.BlockSpec((_BM, _EMB), lambda i: (i, 0)),
        out_shape=jax.ShapeDtypeStruct((_QROWS, _EMB), jnp.float32),
    )(table, table, table, table, w_r)


def _remap_idx(h2, p2, n2):
    def body(h_ref, p_ref, n_ref, ho_ref, po_ref, no_ref):
        for src, dst in ((h_ref, ho_ref), (p_ref, po_ref), (n_ref, no_ref)):
            r = src[...]
            j = ((r >= _QROWS).astype(jnp.int32)
                 + (r >= 2 * _QROWS).astype(jnp.int32)
                 + (r >= 3 * _QROWS).astype(jnp.int32))
            dst[...] = 4 * r - (_N_GRID - 1) * j

    spec = pl.BlockSpec(h2.shape, lambda: (0, 0))
    out = jax.ShapeDtypeStruct(h2.shape, jnp.int32)
    return pl.pallas_call(
        body,
        in_specs=[spec] * 3,
        out_specs=[spec] * 3,
        out_shape=[out] * 3,
    )(h2, p2, n2)


def _edge_z(p_tab, r_vec, h1, tp1, tn1):
    mesh = plsc.VectorSubcoreMesh(core_axis_name="c", subcore_axis_name="s")

    row_t = pltpu.VMEM((_CH, _RDIM), jnp.float32)

    @functools.partial(
        pl.kernel,
        mesh=mesh,
        compiler_params=pltpu.CompilerParams(
            needs_layout_passes=False, use_tc_tiling_on_sc=False),
        out_type=jax.ShapeDtypeStruct((_E,), jnp.float32),
        scratch_types=[
            pltpu.VMEM((_EPW,), jnp.int32),
            pltpu.VMEM((_EPW,), jnp.int32),
            pltpu.VMEM((_EPW,), jnp.int32),
            row_t, row_t, row_t,
            row_t, row_t, row_t,
            pltpu.VMEM((_CH,), jnp.float32),
            pltpu.VMEM((_RDIM,), jnp.float32),
            pltpu.SemaphoreType.DMA,
            pltpu.SemaphoreType.DMA,
        ],
    )
    def kern(p_hbm, r_hbm, h_hbm, tp_hbm, tn_hbm, z_hbm,
             hidx, pidx, nidx, ha, pa, na, hb, pb, nb, zv, rbuf,
             sema, semb):
        wid = lax.axis_index("s") * 2 + lax.axis_index("c")
        ebase = wid * _EPW
        pltpu.sync_copy(r_hbm, rbuf)
        pltpu.sync_copy(h_hbm.at[pl.ds(ebase, _EPW)], hidx)
        pltpu.sync_copy(tp_hbm.at[pl.ds(ebase, _EPW)], pidx)
        pltpu.sync_copy(tn_hbm.at[pl.ds(ebase, _EPW)], nidx)

        def copies(c, hr, pr, nr, sem):
            out = []
            for j in range(_KSUB):
                src = pl.ds(c * _CH + j * _IW, _IW)
                dst = pl.ds(j * _IW, _IW)
                out.append((p_hbm.at[hidx.at[src]], hr.at[dst], sem))
                out.append((p_hbm.at[pidx.at[src]], pr.at[dst], sem))
                out.append((p_hbm.at[nidx.at[src]], nr.at[dst], sem))
            return out

        def issue(c, hr, pr, nr, sem):
            for s, d, sm in copies(c, hr, pr, nr, sem):
                pltpu.async_copy(s, d, sm)

        def drain(c, hr, pr, nr, sem):
            for s, d, sm in copies(c, hr, pr, nr, sem):
                pltpu.make_async_copy(s, d, sm).wait()

        def compute(c, hr, pr, nr):
            def group(g, carry2):
                lane = lax.iota(jnp.int32, 16)
                ridx = lane + g * 16
                g1 = jnp.zeros((16,), jnp.float32)
                g2 = jnp.zeros((16,), jnp.float32)
                cr = jnp.zeros((16,), jnp.float32)
                for d in range(_RDIM):
                    cidx = (lane + d) & (_RDIM - 1)
                    hd = plsc.load_gather(hr, [ridx, cidx])
                    pd = plsc.load_gather(pr, [ridx, cidx])
                    nd = plsc.load_gather(nr, [ridx, cidx])
                    rv = plsc.load_gather(rbuf, [cidx])
                    u = hd - pd
                    v = hd - nd
                    g1 = g1 + u * u
                    g2 = g2 + v * v
                    cr = cr + rv * (u - v)
                zv[pl.ds(g * 16, 16)] = g1 - g2 + cr + cr
                return carry2

            lax.fori_loop(0, _CH // 16, group, 0)
            pltpu.sync_copy(zv, z_hbm.at[pl.ds(ebase + c * _CH, _CH)])

        issue(0, ha, pa, na, sema)

        def pair(k, carry):
            c0 = 2 * k
            issue(c0 + 1, hb, pb, nb, semb)
            drain(c0, ha, pa, na, sema)
            compute(c0, ha, pa, na)
            issue(c0 + 2, ha, pa, na, sema)
            drain(c0 + 1, hb, pb, nb, semb)
            compute(c0 + 1, hb, pb, nb)
            return carry

        lax.fori_loop(0, (_NCHUNK - 1) // 2, pair, 0)
        drain(_NCHUNK - 1, ha, pa, na, sema)
        compute(_NCHUNK - 1, ha, pa, na)

    return kern(p_tab, r_vec, h1, tp1, tn1)


def _softplus_sum(z2d):
    def body(z_ref, o_ref):
        x = z_ref[...]
        sp = jnp.maximum(x, 0.0) + jnp.log1p(jnp.exp(-jnp.abs(x)))
        o_ref[...] = jnp.sum(sp)[None, None]

    return pl.pallas_call(
        body,
        in_specs=[pl.BlockSpec(z2d.shape, lambda: (0, 0))],
        out_specs=pl.BlockSpec((1, 1), lambda: (0, 0)),
        out_shape=jax.ShapeDtypeStruct((1, 1), jnp.float32),
    )(z2d)


def kernel(city_grid_embedding, graph_relation_embed, graph_W_R,
           h, t_pos, t_neg, city_id, relation):
    w_r = graph_W_R[relation]
    r_embed = graph_relation_embed[relation]

    p_tab = _project(city_grid_embedding, w_r).reshape(_N_GRID, _RDIM)

    h2, p2, n2 = (x.astype(jnp.int32).reshape(_E // 128, 128)
                  for x in (h, t_pos, t_neg))
    hm, pm, nm = _remap_idx(h2, p2, n2)

    z = _edge_z(p_tab, r_embed,
                hm.reshape(_E), pm.reshape(_E), nm.reshape(_E))

    loss = _softplus_sum(z.reshape(_E // 128, 128))
    return loss[0, 0]
